# baseline (speedup 1.0000x reference)
import functools

import jax
import jax.numpy as jnp
from jax import lax
from jax.experimental import pallas as pl
from jax.experimental.pallas import tpu as pltpu
from jax.experimental.pallas import tpu_sc as plsc

_CHUNK_D = 8
_NCHUNK = 4


def _zero_span(buf, w0):
    z = jnp.zeros((16,), jnp.float32)

    def body(r, carry):
        dd = r // 32
        hh = r % 32
        buf[dd, hh, pl.ds(w0, 16)] = z
        buf[dd, hh, pl.ds(w0 + 16, 16)] = z
        return carry

    lax.fori_loop(0, _CHUNK_D * 32, body, 0)


def _sc_body(x_hbm, o_hbm, buf0, buf1, sem0, sem1):
    c = lax.axis_index("c")
    s = lax.axis_index("s")
    wid = s * 2 + c
    b = wid // 16
    pd = (wid // 4) % 4
    ph = wid % 4
    d0 = pd * 32
    h0 = ph * 32

    bufs = (buf0, buf1)
    sems = (sem0, sem1)

    def load(k):
        src = x_hbm.at[b, pl.ds(d0 + k * _CHUNK_D, _CHUNK_D), pl.ds(h0, 32), :]
        return pltpu.async_copy(src, bufs[k % 2], sems[k % 2])

    copies = [load(0)]
    for k in range(_NCHUNK):
        if k + 1 < _NCHUNK:
            copies.append(load(k + 1))
        copies[k].wait()
        buf = bufs[k % 2]

        @pl.when((pd == 1) & (ph == 2))
        def _():
            _zero_span(buf, 32)

        @pl.when((pd == 2) & (ph == 1))
        def _():
            _zero_span(buf, 64)

        @pl.when((pd == 2) & (ph == 2))
        def _():
            _zero_span(buf, 0)

        dst = o_hbm.at[b, pl.ds(d0 + k * _CHUNK_D, _CHUNK_D), pl.ds(h0, 32), :]
        pltpu.sync_copy(buf, dst)


def kernel(x):
    B = x.shape[0]
    x4 = x.reshape(B, 128, 128, 128)
    mesh = plsc.VectorSubcoreMesh(core_axis_name="c", subcore_axis_name="s")
    run = functools.partial(
        pl.kernel,
        mesh=mesh,
        out_type=jax.ShapeDtypeStruct((B, 128, 128, 128), jnp.float32),
        scratch_types=[
            pltpu.VMEM((_CHUNK_D, 32, 128), jnp.float32),
            pltpu.VMEM((_CHUNK_D, 32, 128), jnp.float32),
            pltpu.SemaphoreType.DMA,
            pltpu.SemaphoreType.DMA,
        ],
    )(_sc_body)
    out = run(x4)
    return out.reshape(x.shape)


# speedup vs baseline: 1.0364x; 1.0364x over previous
import functools

import jax
import jax.numpy as jnp
from jax import lax
from jax.experimental import pallas as pl
from jax.experimental.pallas import tpu as pltpu
from jax.experimental.pallas import tpu_sc as plsc

_ROWS = 128
_NCH = 8
_NBUF = 7


def _zero_span(buf, r0, w0):
    z = jnp.zeros((16,), jnp.float32)

    def body(i, carry):
        buf[r0 + i, pl.ds(w0, 16)] = z
        buf[r0 + i, pl.ds(w0 + 16, 16)] = z
        return carry

    lax.fori_loop(0, 32, body, 0)


def _sc_body(x_hbm, o_hbm, *scratch):
    bufs = scratch[:_NBUF]
    lsem = scratch[_NBUF:2 * _NBUF]
    ssem = scratch[2 * _NBUF:]

    c = lax.axis_index("c")
    s = lax.axis_index("s")
    wid = s * 2 + c
    base = wid * (_NCH * _ROWS)
    pdq = (wid // 4) % 4

    def start_load(k):
        src = x_hbm.at[pl.ds(base + k * _ROWS, _ROWS), :]
        return pltpu.async_copy(src, bufs[k % _NBUF], lsem[k % _NBUF])

    loads = [start_load(k) for k in range(_NBUF)]
    stores = [None] * _NCH
    for k in range(_NCH):
        loads[k].wait()
        buf = bufs[k % _NBUF]

        @pl.when(pdq == 1)
        def _():
            _zero_span(buf, 64, 32)

        @pl.when(pdq == 2)
        def _():
            _zero_span(buf, 32, 64)
            _zero_span(buf, 64, 0)

        dst = o_hbm.at[pl.ds(base + k * _ROWS, _ROWS), :]
        stores[k] = pltpu.async_copy(buf, dst, ssem[k % _NBUF])
        if k + _NBUF < _NCH:
            stores[k].wait()
            loads.append(start_load(k + _NBUF))
    for k in range(_NCH - _NBUF, _NCH):
        stores[k].wait()


def kernel(x):
    B = x.shape[0]
    nrows = B * 128 * 128
    x2 = x.reshape(nrows, 128)
    mesh = plsc.VectorSubcoreMesh(core_axis_name="c", subcore_axis_name="s")
    run = functools.partial(
        pl.kernel,
        mesh=mesh,
        out_type=jax.ShapeDtypeStruct((nrows, 128), jnp.float32),
        scratch_types=(
            [pltpu.VMEM((_ROWS, 128), jnp.float32)] * _NBUF
            + [pltpu.SemaphoreType.DMA] * (2 * _NBUF)
        ),
    )(_sc_body)
    out = run(x2)
    return out.reshape(x.shape)


# speedup vs baseline: 1.1012x; 1.0625x over previous
import functools

import jax
import jax.numpy as jnp
from jax import lax
from jax.experimental import pallas as pl
from jax.experimental.pallas import tpu as pltpu
from jax.experimental.pallas import tpu_sc as plsc

_ROWS = 256
_NCH = 4
_NBUF = 3


def _zero_span(buf, r0, w0):
    z = jnp.zeros((16,), jnp.float32)

    def body(i, carry):
        buf[r0 + i, pl.ds(w0, 16)] = z
        buf[r0 + i, pl.ds(w0 + 16, 16)] = z
        buf[r0 + 128 + i, pl.ds(w0, 16)] = z
        buf[r0 + 128 + i, pl.ds(w0 + 16, 16)] = z
        return carry

    lax.fori_loop(0, 32, body, 0)


def _sc_body(x_hbm, o_hbm, *scratch):
    bufs = scratch[:_NBUF]
    lsem = scratch[_NBUF:2 * _NBUF]
    ssem = scratch[2 * _NBUF:]

    c = lax.axis_index("c")
    s = lax.axis_index("s")
    wid = s * 2 + c
    base = wid * (_NCH * _ROWS)
    pdq = (wid // 4) % 4

    def start_load(k):
        src = x_hbm.at[pl.ds(base + k * _ROWS, _ROWS), :]
        return pltpu.async_copy(src, bufs[k % _NBUF], lsem[k % _NBUF])

    loads = [start_load(k) for k in range(_NBUF)]
    stores = [None] * _NCH
    for k in range(_NCH):
        loads[k].wait()
        buf = bufs[k % _NBUF]

        @pl.when(pdq == 1)
        def _():
            _zero_span(buf, 64, 32)

        @pl.when(pdq == 2)
        def _():
            _zero_span(buf, 32, 64)
            _zero_span(buf, 64, 0)

        dst = o_hbm.at[pl.ds(base + k * _ROWS, _ROWS), :]
        stores[k] = pltpu.async_copy(buf, dst, ssem[k % _NBUF])
        if k + _NBUF < _NCH:
            stores[k].wait()
            loads.append(start_load(k + _NBUF))
    for k in range(_NCH - _NBUF, _NCH):
        stores[k].wait()


def kernel(x):
    B = x.shape[0]
    nrows = B * 128 * 128
    x2 = x.reshape(nrows, 128)
    mesh = plsc.VectorSubcoreMesh(core_axis_name="c", subcore_axis_name="s")
    run = functools.partial(
        pl.kernel,
        mesh=mesh,
        out_type=jax.ShapeDtypeStruct((nrows, 128), jnp.float32),
        scratch_types=(
            [pltpu.VMEM((_ROWS, 128), jnp.float32)] * _NBUF
            + [pltpu.SemaphoreType.DMA] * (2 * _NBUF)
        ),
    )(_sc_body)
    out = run(x2)
    return out.reshape(x.shape)


# speedup vs baseline: 1.1051x; 1.0035x over previous
import functools

import jax
import jax.numpy as jnp
from jax import lax
from jax.experimental import pallas as pl
from jax.experimental.pallas import tpu as pltpu
from jax.experimental.pallas import tpu_sc as plsc

_ROWS = 256
_NCH = 4
_NBUF = 3


def _zero_span(buf, r0, w0):
    z = jnp.zeros((16,), jnp.float32)

    def body(i, carry):
        buf[r0 + i, pl.ds(w0, 16)] = z
        buf[r0 + i, pl.ds(w0 + 16, 16)] = z
        buf[r0 + 128 + i, pl.ds(w0, 16)] = z
        buf[r0 + 128 + i, pl.ds(w0 + 16, 16)] = z
        return carry

    lax.fori_loop(0, 32, body, 0)


def _sc_body(x_hbm, o_hbm, *scratch):
    bufs = scratch[:_NBUF]
    lsem = scratch[_NBUF:2 * _NBUF]
    ssem = scratch[2 * _NBUF:]

    c = lax.axis_index("c")
    s = lax.axis_index("s")
    wid = c * 16 + s
    base = wid * (_NCH * _ROWS)
    pdq = (wid // 4) % 4

    def start_load(k):
        src = x_hbm.at[pl.ds(base + k * _ROWS, _ROWS), :]
        return pltpu.async_copy(src, bufs[k % _NBUF], lsem[k % _NBUF])

    loads = [start_load(k) for k in range(_NBUF)]
    stores = [None] * _NCH
    for k in range(_NCH):
        loads[k].wait()
        buf = bufs[k % _NBUF]

        @pl.when(pdq == 1)
        def _():
            _zero_span(buf, 64, 32)

        @pl.when(pdq == 2)
        def _():
            _zero_span(buf, 32, 64)
            _zero_span(buf, 64, 0)

        dst = o_hbm.at[pl.ds(base + k * _ROWS, _ROWS), :]
        stores[k] = pltpu.async_copy(buf, dst, ssem[k % _NBUF])
        if k + _NBUF < _NCH:
            stores[k].wait()
            loads.append(start_load(k + _NBUF))
    for k in range(_NCH - _NBUF, _NCH):
        stores[k].wait()


def kernel(x):
    B = x.shape[0]
    nrows = B * 128 * 128
    x2 = x.reshape(nrows, 128)
    mesh = plsc.VectorSubcoreMesh(core_axis_name="c", subcore_axis_name="s")
    run = functools.partial(
        pl.kernel,
        mesh=mesh,
        out_type=jax.ShapeDtypeStruct((nrows, 128), jnp.float32),
        scratch_types=(
            [pltpu.VMEM((_ROWS, 128), jnp.float32)] * _NBUF
            + [pltpu.SemaphoreType.DMA] * (2 * _NBUF)
        ),
    )(_sc_body)
    out = run(x2)
    return out.reshape(x.shape)


# speedup vs baseline: 1.1066x; 1.0014x over previous
import functools

import jax
import jax.numpy as jnp
from jax import lax
from jax.experimental import pallas as pl
from jax.experimental.pallas import tpu as pltpu
from jax.experimental.pallas import tpu_sc as plsc

_CHUNK_ROWS = (128, 384, 384, 128)
_CHUNK_OFF = (0, 128, 512, 896)
_CHUNK_BUF = (0, 1, 2, 0)


def _zero_patch(buf, nd, h0, w0):
    z = jnp.zeros((16,), jnp.float32)

    def body(i, carry):
        for j in range(nd):
            buf[j * 128 + h0 + i, pl.ds(w0, 16)] = z
            buf[j * 128 + h0 + i, pl.ds(w0 + 16, 16)] = z
        return carry

    lax.fori_loop(0, 32, body, 0)


def _sc_body(x_hbm, o_hbm, *scratch):
    bufs = scratch[:3]
    lsem = scratch[3:6]
    ssem = scratch[6:9]

    c = lax.axis_index("c")
    s = lax.axis_index("s")
    wid = c * 16 + s
    base = wid * 1024
    pdq = (wid // 4) % 4

    def start_load(k):
        b = _CHUNK_BUF[k]
        src = x_hbm.at[pl.ds(base + _CHUNK_OFF[k], _CHUNK_ROWS[k]), :]
        return pltpu.async_copy(src, bufs[b], lsem[b])

    loads = [start_load(k) for k in range(3)]
    stores = [None] * 4
    for k in range(4):
        loads[k].wait()
        b = _CHUNK_BUF[k]
        buf = bufs[b]
        nd = _CHUNK_ROWS[k] // 128

        @pl.when(pdq == 1)
        def _(buf=buf, nd=nd):
            _zero_patch(buf, nd, 64, 32)

        @pl.when(pdq == 2)
        def _(buf=buf, nd=nd):
            _zero_patch(buf, nd, 32, 64)
            _zero_patch(buf, nd, 64, 0)

        dst = o_hbm.at[pl.ds(base + _CHUNK_OFF[k], _CHUNK_ROWS[k]), :]
        stores[k] = pltpu.async_copy(buf, dst, ssem[b])
        if k == 0:
            stores[0].wait()
            loads.append(start_load(3))
    for k in range(1, 4):
        stores[k].wait()


def kernel(x):
    B = x.shape[0]
    nrows = B * 128 * 128
    x2 = x.reshape(nrows, 128)
    mesh = plsc.VectorSubcoreMesh(core_axis_name="c", subcore_axis_name="s")
    run = functools.partial(
        pl.kernel,
        mesh=mesh,
        out_type=jax.ShapeDtypeStruct((nrows, 128), jnp.float32),
        scratch_types=(
            [
                pltpu.VMEM((128, 128), jnp.float32),
                pltpu.VMEM((384, 128), jnp.float32),
                pltpu.VMEM((384, 128), jnp.float32),
            ]
            + [pltpu.SemaphoreType.DMA] * 6
        ),
    )(_sc_body)
    out = run(x2)
    return out.reshape(x.shape)
